# baseline (speedup 1.0000x reference)
import functools

import jax
import jax.numpy as jnp
from jax import lax
from jax.experimental import pallas as pl
from jax.experimental.pallas import tpu as pltpu
from jax.experimental.pallas import tpu_sc as plsc

KMERS = 1024
LANES = 16
NUM_WORKERS = 32
SEQ = 200
BATCH = 16384
COLS_PER_TILE = BATCH // NUM_WORKERS
CHUNK_COLS = 128
N_CHUNKS = COLS_PER_TILE // CHUNK_COLS

_MESH = plsc.VectorSubcoreMesh(core_axis_name="c", subcore_axis_name="s")


@functools.partial(
    pl.kernel,
    mesh=_MESH,
    out_type=jax.ShapeDtypeStruct((SEQ, BATCH), jnp.float32),
    scratch_types=[
        pltpu.VMEM((KMERS,), jnp.float32),
        pltpu.VMEM((KMERS,), jnp.float32),
        pltpu.VMEM((SEQ, CHUNK_COLS), jnp.int32),
        pltpu.VMEM((SEQ, CHUNK_COLS), jnp.int32),
        pltpu.VMEM((SEQ, CHUNK_COLS), jnp.float32),
        pltpu.VMEM((SEQ, CHUNK_COLS), jnp.float32),
        pltpu.SemaphoreType.DMA,
        pltpu.SemaphoreType.DMA,
        pltpu.SemaphoreType.DMA,
        pltpu.SemaphoreType.DMA,
    ],
    compiler_params=pltpu.CompilerParams(
        needs_layout_passes=False, use_tc_tiling_on_sc=True),
)
def _sc_lookup(idx_hbm, table_hbm, out_hbm, tab_v, exp_v, i0, i1, o0, o1,
               si0, si1, so0, so1):
    wid = lax.axis_index("s") * 2 + lax.axis_index("c")
    col_base = wid * COLS_PER_TILE

    bufs = ((i0, o0, si0, so0), (i1, o1, si1, so1))

    for b in range(2):
        pltpu.async_copy(
            idx_hbm.at[:, pl.ds(col_base + b * CHUNK_COLS, CHUNK_COLS)],
            bufs[b][0], bufs[b][2])

    pltpu.sync_copy(table_hbm, tab_v)

    @plsc.parallel_loop(0, KMERS // LANES, unroll=4)
    def _exp(i):
        sl = pl.ds(i * LANES, LANES)
        exp_v[sl] = jnp.exp(tab_v[sl])

    def outer(p, carry):
        for b in range(2):
            ib, ob, si, so = bufs[b]
            c = p * 2 + b
            coff = col_base + c * CHUNK_COLS
            hbm_cols = pl.ds(coff, CHUNK_COLS)
            pltpu.make_async_copy(idx_hbm.at[:, hbm_cols], ib, si).wait()

            @pl.when(p > 0)
            def _():
                pltpu.make_async_copy(ob, out_hbm.at[:, hbm_cols], so).wait()

            @plsc.parallel_loop(0, SEQ)
            def _row(r):
                for v in range(CHUNK_COLS // LANES):
                    sl = pl.ds(v * LANES, LANES)
                    ob[r, sl] = plsc.load_gather(exp_v, [ib[r, sl]])

            pltpu.async_copy(ob, out_hbm.at[:, hbm_cols], so)

            @pl.when(c + 2 < N_CHUNKS)
            def _():
                pltpu.async_copy(
                    idx_hbm.at[:, pl.ds(coff + 2 * CHUNK_COLS, CHUNK_COLS)],
                    ib, si)
        return carry

    lax.fori_loop(0, N_CHUNKS // 2, outer, 0)

    for b in range(2):
        ib, ob, si, so = bufs[b]
        coff = col_base + (N_CHUNKS - 2 + b) * CHUNK_COLS
        pltpu.make_async_copy(ob, out_hbm.at[:, pl.ds(coff, CHUNK_COLS)],
                              so).wait()


def kernel(encoded_parents, masks, kmer_embedding):
    del masks
    idx_t = encoded_parents.T.astype(jnp.int32)
    table = kmer_embedding.reshape(-1).astype(jnp.float32)
    return _sc_lookup(idx_t, table).T


# speedup vs baseline: 1.0013x; 1.0013x over previous
import functools

import jax
import jax.numpy as jnp
from jax import lax
from jax.experimental import pallas as pl
from jax.experimental.pallas import tpu as pltpu
from jax.experimental.pallas import tpu_sc as plsc

KMERS = 1024
LANES = 16
NUM_WORKERS = 32
SEQ = 200
BATCH = 16384
COLS_PER_TILE = BATCH // NUM_WORKERS
CHUNK_COLS = 128
N_CHUNKS = COLS_PER_TILE // CHUNK_COLS
HALVES = ((0, 96), (96, 104))

_MESH = plsc.VectorSubcoreMesh(core_axis_name="c", subcore_axis_name="s")


@functools.partial(
    pl.kernel,
    mesh=_MESH,
    out_type=jax.ShapeDtypeStruct((SEQ, BATCH), jnp.float32),
    scratch_types=[
        pltpu.VMEM((KMERS,), jnp.float32),
        pltpu.VMEM((KMERS,), jnp.float32),
        pltpu.VMEM((SEQ, CHUNK_COLS), jnp.int32),
        pltpu.VMEM((SEQ, CHUNK_COLS), jnp.int32),
        pltpu.VMEM((SEQ, CHUNK_COLS), jnp.float32),
        pltpu.VMEM((SEQ, CHUNK_COLS), jnp.float32),
        pltpu.SemaphoreType.DMA,
        pltpu.SemaphoreType.DMA,
        pltpu.SemaphoreType.DMA,
        pltpu.SemaphoreType.DMA,
        pltpu.SemaphoreType.DMA,
        pltpu.SemaphoreType.DMA,
        pltpu.SemaphoreType.DMA,
        pltpu.SemaphoreType.DMA,
    ],
    compiler_params=pltpu.CompilerParams(
        needs_layout_passes=False, use_tc_tiling_on_sc=True),
)
def _sc_lookup(idx_hbm, table_hbm, out_hbm, tab_v, exp_v, i0, i1, o0, o1,
               si00, si01, si10, si11, so00, so01, so10, so11):
    wid = lax.axis_index("s") * 2 + lax.axis_index("c")
    col_base = wid * COLS_PER_TILE

    bufs = ((i0, o0, (si00, si01), (so00, so01)),
            (i1, o1, (si10, si11), (so10, so11)))

    def idx_copy(coff, ib, sems, h):
        r0, nr = HALVES[h]
        return pltpu.make_async_copy(
            idx_hbm.at[pl.ds(r0, nr), pl.ds(coff, CHUNK_COLS)],
            ib.at[pl.ds(r0, nr), :], sems[h])

    def out_copy(coff, ob, sems, h):
        r0, nr = HALVES[h]
        return pltpu.make_async_copy(
            ob.at[pl.ds(r0, nr), :],
            out_hbm.at[pl.ds(r0, nr), pl.ds(coff, CHUNK_COLS)], sems[h])

    for b in range(2):
        ib, _, sin, _ = bufs[b]
        for h in range(2):
            idx_copy(col_base + b * CHUNK_COLS, ib, sin, h).start()

    pltpu.sync_copy(table_hbm, tab_v)

    @plsc.parallel_loop(0, KMERS // LANES, unroll=4)
    def _exp(i):
        sl = pl.ds(i * LANES, LANES)
        exp_v[sl] = jnp.exp(tab_v[sl])

    def outer(p, carry):
        for b in range(2):
            ib, ob, sin, sout = bufs[b]
            c = p * 2 + b
            coff = col_base + c * CHUNK_COLS
            for h in range(2):
                r0, nr = HALVES[h]
                idx_copy(coff, ib, sin, h).wait()

                @pl.when(p > 0)
                def _():
                    out_copy(coff, ob, sout, h).wait()

                @plsc.parallel_loop(r0, r0 + nr)
                def _row(r):
                    for v in range(CHUNK_COLS // LANES):
                        sl = pl.ds(v * LANES, LANES)
                        ob[r, sl] = plsc.load_gather(exp_v, [ib[r, sl]])

                out_copy(coff, ob, sout, h).start()

                @pl.when(c + 2 < N_CHUNKS)
                def _():
                    idx_copy(coff + 2 * CHUNK_COLS, ib, sin, h).start()
        return carry

    lax.fori_loop(0, N_CHUNKS // 2, outer, 0)

    for b in range(2):
        ib, ob, sin, sout = bufs[b]
        coff = col_base + (N_CHUNKS - 2 + b) * CHUNK_COLS
        for h in range(2):
            out_copy(coff, ob, sout, h).wait()


def kernel(encoded_parents, masks, kmer_embedding):
    del masks
    idx_t = encoded_parents.T.astype(jnp.int32)
    table = kmer_embedding.reshape(-1).astype(jnp.float32)
    return _sc_lookup(idx_t, table).T
